# Initial kernel scaffold; baseline (speedup 1.0000x reference)
import jax
import jax.numpy as jnp
from jax.experimental import pallas as pl


def kernel(e_ids, r_ids, entity_embeddings, relation_embeddings):
    raise NotImplementedError("write your pallas kernel here")


# speedup vs baseline: 1.4344x; 1.4344x over previous
import jax
import jax.numpy as jnp
from jax import lax
from jax.experimental import pallas as pl
from jax.experimental.pallas import tpu as pltpu
from jax.experimental.pallas import tpu_sc as plsc

_NUM_ENTITIES = 100000
_NUM_RELATIONS = 1000
_DIM = 128
_BATCH = 4096

_info = plsc.get_sparse_core_info()
_NC, _NS = _info.num_cores, _info.num_subcores
_NW = _NC * _NS
_BPW = _BATCH // _NW

_mesh = plsc.VectorSubcoreMesh(core_axis_name="c", subcore_axis_name="s")


@jax.jit
def _lookup_concat(e_ids, r_ids, entity_embeddings, relation_embeddings):
    @pl.kernel(
        out_type=jax.ShapeDtypeStruct((_BATCH, 2 * _DIM), jnp.float32),
        mesh=_mesh,
        scratch_types=[
            pltpu.VMEM((_BPW,), jnp.int32),
            pltpu.VMEM((_BPW,), jnp.int32),
            pltpu.VMEM((_BPW, _DIM), jnp.float32),
            pltpu.VMEM((_BPW, _DIM), jnp.float32),
            pltpu.SemaphoreType.DMA,
            pltpu.SemaphoreType.DMA,
        ],
    )
    def k(e_hbm, r_hbm, ent_hbm, rel_hbm, out_hbm,
          eidx_v, ridx_v, erows_v, rrows_v, sem_e, sem_r):
        wid = lax.axis_index("s") * _NC + lax.axis_index("c")
        base = wid * _BPW
        pltpu.sync_copy(e_hbm.at[pl.ds(base, _BPW)], eidx_v)
        pltpu.sync_copy(r_hbm.at[pl.ds(base, _BPW)], ridx_v)
        cp_e = pltpu.async_copy(ent_hbm.at[eidx_v], erows_v, sem_e)
        cp_r = pltpu.async_copy(rel_hbm.at[ridx_v], rrows_v, sem_r)
        cp_e.wait()
        cp_r.wait()
        pltpu.sync_copy(erows_v, out_hbm.at[pl.ds(base, _BPW), pl.ds(0, _DIM)])
        pltpu.sync_copy(rrows_v, out_hbm.at[pl.ds(base, _BPW), pl.ds(_DIM, _DIM)])

    return k(e_ids, r_ids, entity_embeddings, relation_embeddings)


def kernel(e_ids, r_ids, entity_embeddings, relation_embeddings):
    return _lookup_concat(e_ids, r_ids, entity_embeddings, relation_embeddings)


# speedup vs baseline: 1.4443x; 1.0069x over previous
import jax
import jax.numpy as jnp
from jax import lax
from jax.experimental import pallas as pl
from jax.experimental.pallas import tpu as pltpu
from jax.experimental.pallas import tpu_sc as plsc

_NUM_ENTITIES = 100000
_NUM_RELATIONS = 1000
_DIM = 128
_BATCH = 4096

_info = plsc.get_sparse_core_info()
_NC, _NS = _info.num_cores, _info.num_subcores
_NW = _NC * _NS
_BPW = _BATCH // _NW

_mesh = plsc.VectorSubcoreMesh(core_axis_name="c", subcore_axis_name="s")


@jax.jit
def _lookup_concat(e_ids, r_ids, entity_embeddings, relation_embeddings):
    @pl.kernel(
        out_type=jax.ShapeDtypeStruct((_BATCH, 2 * _DIM), jnp.float32),
        mesh=_mesh,
        scratch_types=[
            pltpu.VMEM((_BPW,), jnp.int32),
            pltpu.VMEM((_BPW,), jnp.int32),
            pltpu.VMEM((_BPW, _DIM), jnp.float32),
            pltpu.VMEM((_BPW, _DIM), jnp.float32),
            pltpu.SemaphoreType.DMA,
            pltpu.SemaphoreType.DMA,
            pltpu.SemaphoreType.DMA,
            pltpu.SemaphoreType.DMA,
        ],
    )
    def k(e_hbm, r_hbm, ent_hbm, rel_hbm, out_hbm,
          eidx_v, ridx_v, erows_v, rrows_v, sem_e, sem_r, sem_we, sem_wr):
        wid = lax.axis_index("s") * _NC + lax.axis_index("c")
        base = wid * _BPW
        cp_ei = pltpu.async_copy(e_hbm.at[pl.ds(base, _BPW)], eidx_v, sem_e)
        cp_ri = pltpu.async_copy(r_hbm.at[pl.ds(base, _BPW)], ridx_v, sem_r)
        cp_ei.wait()
        cp_e = pltpu.async_copy(ent_hbm.at[eidx_v], erows_v, sem_e)
        cp_ri.wait()
        cp_r = pltpu.async_copy(rel_hbm.at[ridx_v], rrows_v, sem_r)
        cp_e.wait()
        cp_we = pltpu.async_copy(
            erows_v, out_hbm.at[pl.ds(base, _BPW), pl.ds(0, _DIM)], sem_we)
        cp_r.wait()
        cp_wr = pltpu.async_copy(
            rrows_v, out_hbm.at[pl.ds(base, _BPW), pl.ds(_DIM, _DIM)], sem_wr)
        cp_we.wait()
        cp_wr.wait()

    return k(e_ids, r_ids, entity_embeddings, relation_embeddings)


def kernel(e_ids, r_ids, entity_embeddings, relation_embeddings):
    return _lookup_concat(e_ids, r_ids, entity_embeddings, relation_embeddings)
